# Initial kernel scaffold; baseline (speedup 1.0000x reference)
import jax
import jax.numpy as jnp
from jax.experimental import pallas as pl


def kernel(z, m, W):
    raise NotImplementedError("write your pallas kernel here")


# speedup vs baseline: 56.1227x; 56.1227x over previous
import jax
import jax.numpy as jnp
from jax.experimental import pallas as pl
from jax.experimental.pallas import tpu as pltpu

_B, _NZ, _NM, _FIN, _FQK = 8, 256, 256, 64, 32
_TI = 32


def _body(z_ref, m_ref, w_ref, out_ref):
    z_t = z_ref[0]
    m_b = m_ref[0]
    W = w_ref[...]
    scale = _FQK ** -0.5
    q = jax.lax.dot_general(z_t, W[:_FQK, :], (((1,), (1,)), ((), ())),
                            preferred_element_type=jnp.float32) * scale
    k = jax.lax.dot_general(m_b, W[_FQK:, :], (((1,), (1,)), ((), ())),
                            preferred_element_type=jnp.float32)
    aw = jax.lax.dot_general(q, k, (((1,), (1,)), ((), ())),
                             preferred_element_type=jnp.float32)
    ex = jnp.exp(aw)
    w = ex / jnp.sum(ex, axis=1, keepdims=True)

    diff = z_t[:, None, :] - m_b[None, :, :]
    out_ref[0] = diff * diff * w[:, :, None]


def kernel(z, m, W):
    out = pl.pallas_call(
        _body,
        grid=(_B, _NZ // _TI),
        in_specs=[
            pl.BlockSpec((1, _TI, _FIN), lambda b, t: (b, t, 0)),
            pl.BlockSpec((1, _NM, _FIN), lambda b, t: (b, 0, 0)),
            pl.BlockSpec((2 * _FQK, _FIN), lambda b, t: (0, 0)),
        ],
        out_specs=pl.BlockSpec((1, _TI, _NM, _FIN), lambda b, t: (b, t, 0, 0)),
        out_shape=jax.ShapeDtypeStruct((_B, _NZ, _NM, _FIN), jnp.float32),
    )(z, m, W)
    return out.reshape(_B * _NZ * _NM, _FIN)
